# baseline (speedup 1.0000x reference)
import jax
import jax.numpy as jnp
from jax.experimental import pallas as pl

_BANK = 100000
_BATCH = 16384
_D = 128
_BLK = 8192
_NFEAT_BLKS = _BATCH // _BLK
_NBLKS = (_BANK + _BLK - 1) // _BLK


def _body(feats_ref, bank_ref, out_ref):
    i = pl.program_id(0)

    @pl.when(i < _NFEAT_BLKS)
    def _():
        x = feats_ref[...]
        n2 = jnp.sum(x * x, axis=1, keepdims=True)
        out_ref[...] = x * jax.lax.rsqrt(jnp.maximum(n2, 1e-24))

    @pl.when(i >= _NFEAT_BLKS)
    def _():
        out_ref[...] = bank_ref[...]


def kernel(feats, bank):
    return pl.pallas_call(
        _body,
        grid=(_NBLKS,),
        in_specs=[
            pl.BlockSpec((_BLK, _D), lambda i: (jnp.minimum(i, _NFEAT_BLKS - 1), 0)),
            pl.BlockSpec((_BLK, _D), lambda i: (jnp.maximum(i, _NFEAT_BLKS), 0)),
        ],
        out_specs=pl.BlockSpec((_BLK, _D), lambda i: (i, 0)),
        out_shape=jax.ShapeDtypeStruct((_BANK, _D), jnp.float32),
    )(feats, bank)
